# baseline (speedup 1.0000x reference)
import functools

import numpy as np
import jax
import jax.numpy as jnp
from jax import lax
from jax.experimental import pallas as pl
from jax.experimental.pallas import tpu as pltpu
from jax.experimental.pallas import tpu_sc as plsc

NUM_TRAIN = 256
A = 20000
G = 100
CHUNK = 800
NCHUNK = A // CHUNK
NSTREAM = 16384
NEGSZ = 20272


def _mt_streams(nseeds: int, n: int) -> np.ndarray:
    u32 = np.uint32

    def twist(key):
        def f(cur, nxt, far):
            y = (cur & u32(0x80000000)) | (nxt & u32(0x7FFFFFFF))
            v = far ^ (y >> u32(1))
            return np.where((y & u32(1)) == 1, v ^ u32(0x9908B0DF), v)

        new = np.empty_like(key)
        new[:227] = f(key[:227], key[1:228], key[397:624])
        new[227:454] = f(key[227:454], key[228:455], new[0:227])
        new[454:623] = f(key[454:623], key[455:624], new[227:396])
        new[623] = f(key[623:624], new[0:1], new[396:397])[0]
        return new

    def temper(y):
        y = y ^ (y >> u32(11))
        y = y ^ ((y << u32(7)) & u32(0x9D2C5680))
        y = y ^ ((y << u32(15)) & u32(0xEFC60000))
        y = y ^ (y >> u32(18))
        return y

    out = np.empty((nseeds, n), dtype=np.uint32)
    nblocks = -(-n // 624)
    for seed in range(nseeds):
        key = np.empty(624, dtype=np.uint64)
        s = seed & 0xFFFFFFFF
        for p in range(624):
            key[p] = s
            s = (1812433253 * (s ^ (s >> 30)) + p + 1) & 0xFFFFFFFF
        key = key.astype(np.uint32)
        blocks = []
        for _ in range(nblocks):
            key = twist(key)
            blocks.append(temper(key))
        out[seed] = np.concatenate(blocks)[:n]
    return out.view(np.int32)


_STREAMS = _mt_streams(8, NSTREAM)


def _stage1_body(a_ref, g_ref, tc_ref, ti_ref, am_ref, mx_scr, am_scr):
    c = pl.program_id(1)
    a = jnp.clip(a_ref[0], 0.0, 1.0)
    ay1, ax1, ay2, ax2 = a[:, 0:1], a[:, 1:2], a[:, 2:3], a[:, 3:4]
    gy1 = g_ref[0, 0:1, :]
    gx1 = g_ref[0, 1:2, :]
    gy2 = g_ref[0, 2:3, :]
    gx2 = g_ref[0, 3:4, :]
    ih = jnp.maximum(jnp.minimum(ay2, gy2) - jnp.maximum(ay1, gy1), 0.0)
    iw = jnp.maximum(jnp.minimum(ax2, gx2) - jnp.maximum(ax1, gx1), 0.0)
    inter = ih * iw
    area_a = (ay2 - ay1) * (ax2 - ax1)
    area_g = (gy2 - gy1) * (gx2 - gx1)
    ov = inter / (area_a + area_g - inter + 1e-8)

    mx = jnp.max(ov, axis=1)
    iotag = lax.broadcasted_iota(jnp.int32, (CHUNK, G), 1)
    ti_c = jnp.min(jnp.where(ov == mx[:, None], iotag, G), axis=1)
    tc_c = jnp.where(mx < 0.3, 0, -1).astype(jnp.int32)
    tc_c = jnp.where(mx >= 0.7, 1, tc_c)
    tc_ref[0, 0, 0, :] = tc_c
    ti_ref[0, 0, 0, :] = ti_c

    cmx = jnp.max(ov, axis=0)
    iotaa = lax.broadcasted_iota(jnp.int32, (CHUNK, G), 0) + c * CHUNK
    cam = jnp.min(jnp.where(ov == cmx[None, :], iotaa, A), axis=0)

    @pl.when(c == 0)
    def _():
        mx_scr[0, :] = jnp.full((128,), -1.0, jnp.float32)
        am_scr[0, :] = jnp.zeros((128,), jnp.int32)

    prev_mx = mx_scr[0, 0:G]
    prev_am = am_scr[0, 0:G]
    upd = cmx > prev_mx
    mx_scr[0, 0:G] = jnp.where(upd, cmx, prev_mx)
    am_scr[0, 0:G] = jnp.where(upd, cam, prev_am)

    @pl.when(c == NCHUNK - 1)
    def _():
        am_ref[0, 0, 0, 0:G] = am_scr[0, 0:G]


def _stage1(anchors, gt_t):
    return pl.pallas_call(
        _stage1_body,
        grid=(8, NCHUNK),
        in_specs=[
            pl.BlockSpec((1, CHUNK, 4), lambda b, c: (b, c, 0)),
            pl.BlockSpec((1, 4, G), lambda b, c: (b, 0, 0)),
        ],
        out_specs=[
            pl.BlockSpec((1, 1, 1, CHUNK), lambda b, c: (b, c, 0, 0)),
            pl.BlockSpec((1, 1, 1, CHUNK), lambda b, c: (b, c, 0, 0)),
            pl.BlockSpec((1, 1, 1, 128), lambda b, c: (b, 0, 0, 0)),
        ],
        out_shape=[
            jax.ShapeDtypeStruct((8, NCHUNK, 1, CHUNK), jnp.int32),
            jax.ShapeDtypeStruct((8, NCHUNK, 1, CHUNK), jnp.int32),
            jax.ShapeDtypeStruct((8, 1, 1, 128), jnp.int32),
        ],
        scratch_shapes=[
            pltpu.VMEM((8, 128), jnp.float32),
            pltpu.VMEM((8, 128), jnp.int32),
        ],
    )(anchors, gt_t)


def _sc_body(tc_hbm, am_hbm, st_hbm, cls_hbm, tc_v, am_v, st_v, pos_v, neg_v):
    cid = lax.axis_index("c")
    sid = lax.axis_index("s")
    wid = sid * 2 + cid

    @pl.when(wid < 8)
    def _():
        b = wid
        with jax.named_scope("sc_dma_in"):
            pltpu.sync_copy(tc_hbm.at[b], tc_v)
            pltpu.sync_copy(am_hbm.at[b], am_v)
            pltpu.sync_copy(st_hbm.at[b], st_v.at[pl.ds(0, NSTREAM)])

        iota16 = lax.iota(jnp.int32, 16)
        one16 = jnp.ones((16,), jnp.int32)

        for k in range(7):
            idx = am_v[pl.ds(k * 16, 16)]
            msk = (iota16 + k * 16) < G
            plsc.store_scatter(tc_v, [idx], one16, mask=msk)

        def _scalar(x):
            return x[0] if getattr(x, "ndim", 0) else x

        def comp_body(ci, carry):
            poff, noff = carry
            for u in range(5):
                cc = ci * 5 + u
                v = tc_v[pl.ds(cc * 16, 16)]
                ids = iota16 + cc * 16
                pm = v == 1
                nm = v == 0
                plsc.store_compressed(pos_v.at[pl.ds(poff, 16)], ids, mask=pm)
                plsc.store_compressed(neg_v.at[pl.ds(noff, 16)], ids, mask=nm)
                poff = poff + _scalar(plsc.all_reduce_population_count(pm))
                noff = noff + _scalar(plsc.all_reduce_population_count(nm))
            return (poff, noff)

        with jax.named_scope("sc_compact"):
            p, m = lax.fori_loop(0, A // 80, comp_body,
                                 (jnp.int32(0), jnp.int32(0)))

        zero16 = jnp.zeros((16,), jnp.int32)

        def scan_accept(i, t):
            msk = i | (i >> 1)
            msk = msk | (msk >> 2)
            msk = msk | (msk >> 4)
            msk = msk | (msk >> 8)
            msk = msk | (msk >> 16)

            def cond(tt):
                v = st_v[pl.ds(tt, 16)] & msk
                return jnp.logical_not(jnp.any(v <= i)) & (tt < NSTREAM)

            t = lax.while_loop(cond, lambda tt: tt + 16, t)
            v = st_v[pl.ds(t, 16)] & msk
            r = plsc.all_reduce_ffs(v <= i)
            r = r[0] if getattr(r, "ndim", 0) else r
            j = v.at[zero16 + r].get(mode="promise_in_bounds")[0]
            return j, t + r + 1

        swap_perm = jnp.where(iota16 == 0, 1, 0)

        def shuffle_phase(buf, n, excess, t):
            def sbody(s, t):
                i = n - 1 - s
                j, t = scan_accept(i, t)
                idxv = jnp.where(iota16 == 0, i, j)
                vals = plsc.load_gather(buf, [idxv])
                swapped = vals.at[swap_perm].get(mode="promise_in_bounds")
                plsc.store_scatter(buf, [idxv], swapped, mask=iota16 < 2)
                return t

            t = lax.fori_loop(0, jnp.where(excess > 0, n - excess, 0),
                              sbody, t)

            def cbody(s, t):
                i = excess - 1 - s
                _, t = scan_accept(i, t)
                return t

            return lax.fori_loop(0, jnp.where(excess > 0, excess - 1, 0),
                                 cbody, t)

        excess_pos = p - NUM_TRAIN // 2
        with jax.named_scope("sc_shuffle_pos"):
            t = shuffle_phase(pos_v, p, excess_pos, jnp.int32(0))
        remaining_pos = p - jnp.maximum(excess_pos, 0)
        desired_neg = NUM_TRAIN - remaining_pos
        excess_neg = m - desired_neg
        with jax.named_scope("sc_shuffle_neg"):
            shuffle_phase(neg_v, m, excess_neg, t)

        kp = excess_pos > 0
        kn = excess_neg > 0

        neg_one16 = jnp.full((16,), -1, jnp.int32)

        def ms_body(ci, carry):
            for u in range(10):
                tc_v[pl.ds((ci * 10 + u) * 16, 16)] = neg_one16
            return carry

        with jax.named_scope("sc_memset"):
            lax.fori_loop(0, A // 160, ms_body, jnp.int32(0))

        pstart = jnp.where(kp, excess_pos, 0)
        pcount = jnp.where(kp, NUM_TRAIN // 2, p)

        def pk_body(k, carry):
            idx = pos_v[pl.ds(pstart + k * 16, 16)]
            msk = (iota16 + k * 16) < pcount
            plsc.store_scatter(tc_v, [idx], one16, mask=msk)
            return carry

        lax.fori_loop(0, (pcount + 15) // 16, pk_body, jnp.int32(0))

        nstart = jnp.where(kn, excess_neg, 0)
        ncount = jnp.where(kn, desired_neg, m)

        def nk_body(k, carry):
            idx = neg_v[pl.ds(nstart + k * 16, 16)]
            msk = (iota16 + k * 16) < ncount
            plsc.store_scatter(tc_v, [idx], jnp.zeros((16,), jnp.int32),
                               mask=msk)
            return carry

        lax.fori_loop(0, (ncount + 15) // 16, nk_body, jnp.int32(0))

        with jax.named_scope("sc_dma_out"):
            pltpu.sync_copy(tc_v, cls_hbm.at[b])


def _stage2(tc0, amax, streams):
    mesh = plsc.VectorSubcoreMesh(core_axis_name="c", subcore_axis_name="s")
    f = functools.partial(
        pl.kernel,
        mesh=mesh,
        compiler_params=pltpu.CompilerParams(needs_layout_passes=False),
        out_type=jax.ShapeDtypeStruct((8, A), jnp.int32),
        scratch_types=[
            pltpu.VMEM((A,), jnp.int32),
            pltpu.VMEM((128,), jnp.int32),
            pltpu.VMEM((NSTREAM + 16,), jnp.int32),
            pltpu.VMEM((A + 16,), jnp.int32),
            pltpu.VMEM((NEGSZ,), jnp.int32),
        ],
    )(_sc_body)
    return f(tc0, amax, streams)


def _stage3_body(a_ref, g_ref, ti_ref, cls_ref, d_ref):
    a = jnp.clip(a_ref[0], 0.0, 1.0)
    ti_c = ti_ref[0, 0]
    cls = cls_ref[0, 0]
    onehot = (ti_c == lax.broadcasted_iota(jnp.int32, (CHUNK, G), 1))
    gm = lax.dot_general(onehot.astype(jnp.float32), g_ref[0],
                         (((1,), (0,)), ((), ())),
                         precision=lax.Precision.HIGHEST,
                         preferred_element_type=jnp.float32)
    ah = jnp.maximum(a[:, 2:3] - a[:, 0:1], 1e-6)
    aw = jnp.maximum(a[:, 3:4] - a[:, 1:2], 1e-6)
    acy = (a[:, 0:1] + a[:, 2:3]) * 0.5
    acx = (a[:, 1:2] + a[:, 3:4]) * 0.5
    gh = jnp.maximum(gm[:, 2:3] - gm[:, 0:1], 1e-6)
    gw = jnp.maximum(gm[:, 3:4] - gm[:, 1:2], 1e-6)
    gcy = (gm[:, 0:1] + gm[:, 2:3]) * 0.5
    gcx = (gm[:, 1:2] + gm[:, 3:4]) * 0.5
    pd = jnp.concatenate(
        [(gcy - acy) / ah, (gcx - acx) / aw,
         jnp.log(gh / ah), jnp.log(gw / aw)], axis=1)
    d_ref[0] = jnp.where(cls > 0, pd, 0.0)


def _stage3(anchors, gt, ti, cls4):
    return pl.pallas_call(
        _stage3_body,
        grid=(8, NCHUNK),
        in_specs=[
            pl.BlockSpec((1, CHUNK, 4), lambda b, c: (b, c, 0)),
            pl.BlockSpec((1, G, 4), lambda b, c: (b, 0, 0)),
            pl.BlockSpec((1, 1, CHUNK, 1), lambda b, c: (b, c, 0, 0)),
            pl.BlockSpec((1, 1, CHUNK, 1), lambda b, c: (b, c, 0, 0)),
        ],
        out_specs=pl.BlockSpec((1, CHUNK, 4), lambda b, c: (b, c, 0)),
        out_shape=jax.ShapeDtypeStruct((8, A, 4), jnp.float32),
    )(anchors, gt, ti, cls4)


def kernel(anchors, true_bboxes):
    gt_t = true_bboxes.transpose(0, 2, 1)
    tc0, ti, amax = _stage1(anchors, gt_t)
    streams = jnp.asarray(_STREAMS)
    classes = _stage2(tc0.reshape(8, A), amax.reshape(8, 128), streams)
    deltas = _stage3(anchors, true_bboxes, ti.reshape(8, NCHUNK, CHUNK, 1),
                     classes.reshape(8, NCHUNK, CHUNK, 1))
    return classes, deltas


# speedup vs baseline: 1.5916x; 1.5916x over previous
import functools

import numpy as np
import jax
import jax.numpy as jnp
from jax import lax
from jax.experimental import pallas as pl
from jax.experimental.pallas import tpu as pltpu
from jax.experimental.pallas import tpu_sc as plsc

NUM_TRAIN = 256
A = 20000
G = 100
CHUNK = 800
NCHUNK = A // CHUNK
NSTREAM = 16384
NEGSZ = 20272


def _mt_streams(nseeds: int, n: int) -> np.ndarray:
    u32 = np.uint32

    def twist(key):
        def f(cur, nxt, far):
            y = (cur & u32(0x80000000)) | (nxt & u32(0x7FFFFFFF))
            v = far ^ (y >> u32(1))
            return np.where((y & u32(1)) == 1, v ^ u32(0x9908B0DF), v)

        new = np.empty_like(key)
        new[:227] = f(key[:227], key[1:228], key[397:624])
        new[227:454] = f(key[227:454], key[228:455], new[0:227])
        new[454:623] = f(key[454:623], key[455:624], new[227:396])
        new[623] = f(key[623:624], new[0:1], new[396:397])[0]
        return new

    def temper(y):
        y = y ^ (y >> u32(11))
        y = y ^ ((y << u32(7)) & u32(0x9D2C5680))
        y = y ^ ((y << u32(15)) & u32(0xEFC60000))
        y = y ^ (y >> u32(18))
        return y

    out = np.empty((nseeds, n), dtype=np.uint32)
    nblocks = -(-n // 624)
    for seed in range(nseeds):
        key = np.empty(624, dtype=np.uint64)
        s = seed & 0xFFFFFFFF
        for p in range(624):
            key[p] = s
            s = (1812433253 * (s ^ (s >> 30)) + p + 1) & 0xFFFFFFFF
        key = key.astype(np.uint32)
        blocks = []
        for _ in range(nblocks):
            key = twist(key)
            blocks.append(temper(key))
        out[seed] = np.concatenate(blocks)[:n]
    return out.view(np.int32)


_STREAMS = _mt_streams(8, NSTREAM)


def _stage1_body(a_ref, g_ref, tc_ref, ti_ref, am_ref, mx_scr, am_scr):
    c = pl.program_id(1)
    a = jnp.clip(a_ref[0], 0.0, 1.0)
    ay1, ax1, ay2, ax2 = a[:, 0:1], a[:, 1:2], a[:, 2:3], a[:, 3:4]
    gy1 = g_ref[0, 0:1, :]
    gx1 = g_ref[0, 1:2, :]
    gy2 = g_ref[0, 2:3, :]
    gx2 = g_ref[0, 3:4, :]
    ih = jnp.maximum(jnp.minimum(ay2, gy2) - jnp.maximum(ay1, gy1), 0.0)
    iw = jnp.maximum(jnp.minimum(ax2, gx2) - jnp.maximum(ax1, gx1), 0.0)
    inter = ih * iw
    area_a = (ay2 - ay1) * (ax2 - ax1)
    area_g = (gy2 - gy1) * (gx2 - gx1)
    ov = inter / (area_a + area_g - inter + 1e-8)

    mx = jnp.max(ov, axis=1)
    iotag = lax.broadcasted_iota(jnp.int32, (CHUNK, G), 1)
    ti_c = jnp.min(jnp.where(ov == mx[:, None], iotag, G), axis=1)
    tc_c = jnp.where(mx < 0.3, 0, -1).astype(jnp.int32)
    tc_c = jnp.where(mx >= 0.7, 1, tc_c)
    tc_ref[0, 0, 0, :] = tc_c
    ti_ref[0, 0, 0, :] = ti_c

    cmx = jnp.max(ov, axis=0)
    iotaa = lax.broadcasted_iota(jnp.int32, (CHUNK, G), 0) + c * CHUNK
    cam = jnp.min(jnp.where(ov == cmx[None, :], iotaa, A), axis=0)

    @pl.when(c == 0)
    def _():
        mx_scr[0, :] = jnp.full((128,), -1.0, jnp.float32)
        am_scr[0, :] = jnp.zeros((128,), jnp.int32)

    prev_mx = mx_scr[0, 0:G]
    prev_am = am_scr[0, 0:G]
    upd = cmx > prev_mx
    mx_scr[0, 0:G] = jnp.where(upd, cmx, prev_mx)
    am_scr[0, 0:G] = jnp.where(upd, cam, prev_am)

    @pl.when(c == NCHUNK - 1)
    def _():
        am_ref[0, 0, 0, 0:G] = am_scr[0, 0:G]


def _stage1(anchors, gt_t):
    return pl.pallas_call(
        _stage1_body,
        grid=(8, NCHUNK),
        in_specs=[
            pl.BlockSpec((1, CHUNK, 4), lambda b, c: (b, c, 0)),
            pl.BlockSpec((1, 4, G), lambda b, c: (b, 0, 0)),
        ],
        out_specs=[
            pl.BlockSpec((1, 1, 1, CHUNK), lambda b, c: (b, c, 0, 0)),
            pl.BlockSpec((1, 1, 1, CHUNK), lambda b, c: (b, c, 0, 0)),
            pl.BlockSpec((1, 1, 1, 128), lambda b, c: (b, 0, 0, 0)),
        ],
        out_shape=[
            jax.ShapeDtypeStruct((8, NCHUNK, 1, CHUNK), jnp.int32),
            jax.ShapeDtypeStruct((8, NCHUNK, 1, CHUNK), jnp.int32),
            jax.ShapeDtypeStruct((8, 1, 1, 128), jnp.int32),
        ],
        scratch_shapes=[
            pltpu.VMEM((8, 128), jnp.float32),
            pltpu.VMEM((8, 128), jnp.int32),
        ],
    )(anchors, gt_t)


def _sc_body(tc_hbm, am_hbm, st_hbm, cls_hbm, tc_v, am_v, st_v, pos_v, neg_v):
    cid = lax.axis_index("c")
    sid = lax.axis_index("s")
    wid = sid * 2 + cid

    @pl.when(wid < 8)
    def _():
        b = wid
        with jax.named_scope("sc_dma_in"):
            pltpu.sync_copy(tc_hbm.at[b], tc_v)
            pltpu.sync_copy(am_hbm.at[b], am_v)
            pltpu.sync_copy(st_hbm.at[b], st_v.at[pl.ds(0, NSTREAM)])

        iota16 = lax.iota(jnp.int32, 16)
        one16 = jnp.ones((16,), jnp.int32)

        for k in range(7):
            idx = am_v[pl.ds(k * 16, 16)]
            msk = (iota16 + k * 16) < G
            plsc.store_scatter(tc_v, [idx], one16, mask=msk)

        def _scalar(x):
            return x[0] if getattr(x, "ndim", 0) else x

        def comp_body(ci, carry):
            poff, noff = carry
            for u in range(5):
                cc = ci * 5 + u
                v = tc_v[pl.ds(cc * 16, 16)]
                ids = iota16 + cc * 16
                pm = v == 1
                nm = v == 0
                plsc.store_compressed(pos_v.at[pl.ds(poff, 16)], ids, mask=pm)
                plsc.store_compressed(neg_v.at[pl.ds(noff, 16)], ids, mask=nm)
                poff = poff + _scalar(plsc.all_reduce_population_count(pm))
                noff = noff + _scalar(plsc.all_reduce_population_count(nm))
            return (poff, noff)

        with jax.named_scope("sc_compact"):
            p, m = lax.fori_loop(0, A // 80, comp_body,
                                 (jnp.int32(0), jnp.int32(0)))

        zero16 = jnp.zeros((16,), jnp.int32)

        def scan_accept(i, t):
            msk = i | (i >> 1)
            msk = msk | (msk >> 2)
            msk = msk | (msk >> 4)
            msk = msk | (msk >> 8)
            msk = msk | (msk >> 16)

            def cond(tt):
                v = st_v[pl.ds(tt, 16)] & msk
                return jnp.logical_not(jnp.any(v <= i)) & (tt < NSTREAM)

            t = lax.while_loop(cond, lambda tt: tt + 16, t)
            v = st_v[pl.ds(t, 16)] & msk
            r = plsc.all_reduce_ffs(v <= i)
            r = r[0] if getattr(r, "ndim", 0) else r
            j = v.at[zero16 + r].get(mode="promise_in_bounds")[0]
            return j, t + r + 1

        swap_perm = jnp.where(iota16 == 0, 1, 0)

        def shuffle_phase(buf, n, excess, t, count_tail):
            def sbody(s, t):
                i = n - 1 - s
                j, t = scan_accept(i, t)
                idxv = jnp.where(iota16 == 0, i, j)
                vals = plsc.load_gather(buf, [idxv])
                swapped = vals.at[swap_perm].get(mode="promise_in_bounds")
                plsc.store_scatter(buf, [idxv], swapped, mask=iota16 < 2)
                return t

            t = lax.fori_loop(0, jnp.where(excess > 0, n - excess, 0),
                              sbody, t)

            if not count_tail:
                return t

            def cbody(s, t):
                i = excess - 1 - s
                _, t = scan_accept(i, t)
                return t

            return lax.fori_loop(0, jnp.where(excess > 0, excess - 1, 0),
                                 cbody, t)

        excess_pos = p - NUM_TRAIN // 2
        with jax.named_scope("sc_shuffle_pos"):
            t = shuffle_phase(pos_v, p, excess_pos, jnp.int32(0), True)
        remaining_pos = p - jnp.maximum(excess_pos, 0)
        desired_neg = NUM_TRAIN - remaining_pos
        excess_neg = m - desired_neg
        with jax.named_scope("sc_shuffle_neg"):
            shuffle_phase(neg_v, m, excess_neg, t, False)

        kp = excess_pos > 0
        kn = excess_neg > 0

        neg_one16 = jnp.full((16,), -1, jnp.int32)

        def ms_body(ci, carry):
            for u in range(10):
                tc_v[pl.ds((ci * 10 + u) * 16, 16)] = neg_one16
            return carry

        with jax.named_scope("sc_memset"):
            lax.fori_loop(0, A // 160, ms_body, jnp.int32(0))

        pstart = jnp.where(kp, excess_pos, 0)
        pcount = jnp.where(kp, NUM_TRAIN // 2, p)

        def pk_body(k, carry):
            idx = pos_v[pl.ds(pstart + k * 16, 16)]
            msk = (iota16 + k * 16) < pcount
            plsc.store_scatter(tc_v, [idx], one16, mask=msk)
            return carry

        lax.fori_loop(0, (pcount + 15) // 16, pk_body, jnp.int32(0))

        nstart = jnp.where(kn, excess_neg, 0)
        ncount = jnp.where(kn, desired_neg, m)

        def nk_body(k, carry):
            idx = neg_v[pl.ds(nstart + k * 16, 16)]
            msk = (iota16 + k * 16) < ncount
            plsc.store_scatter(tc_v, [idx], jnp.zeros((16,), jnp.int32),
                               mask=msk)
            return carry

        lax.fori_loop(0, (ncount + 15) // 16, nk_body, jnp.int32(0))

        with jax.named_scope("sc_dma_out"):
            pltpu.sync_copy(tc_v, cls_hbm.at[b])


def _stage2(tc0, amax, streams):
    mesh = plsc.VectorSubcoreMesh(core_axis_name="c", subcore_axis_name="s")
    f = functools.partial(
        pl.kernel,
        mesh=mesh,
        compiler_params=pltpu.CompilerParams(needs_layout_passes=False),
        out_type=jax.ShapeDtypeStruct((8, A), jnp.int32),
        scratch_types=[
            pltpu.VMEM((A,), jnp.int32),
            pltpu.VMEM((128,), jnp.int32),
            pltpu.VMEM((NSTREAM + 16,), jnp.int32),
            pltpu.VMEM((A + 16,), jnp.int32),
            pltpu.VMEM((NEGSZ,), jnp.int32),
        ],
    )(_sc_body)
    return f(tc0, amax, streams)


def _stage3_body(a_ref, g_ref, ti_ref, cls_ref, d_ref):
    a = jnp.clip(a_ref[0], 0.0, 1.0)
    ti_c = ti_ref[0, 0]
    cls = cls_ref[0, 0]
    onehot = (ti_c == lax.broadcasted_iota(jnp.int32, (CHUNK, G), 1))
    gm = lax.dot_general(onehot.astype(jnp.float32), g_ref[0],
                         (((1,), (0,)), ((), ())),
                         precision=lax.Precision.HIGHEST,
                         preferred_element_type=jnp.float32)
    ah = jnp.maximum(a[:, 2:3] - a[:, 0:1], 1e-6)
    aw = jnp.maximum(a[:, 3:4] - a[:, 1:2], 1e-6)
    acy = (a[:, 0:1] + a[:, 2:3]) * 0.5
    acx = (a[:, 1:2] + a[:, 3:4]) * 0.5
    gh = jnp.maximum(gm[:, 2:3] - gm[:, 0:1], 1e-6)
    gw = jnp.maximum(gm[:, 3:4] - gm[:, 1:2], 1e-6)
    gcy = (gm[:, 0:1] + gm[:, 2:3]) * 0.5
    gcx = (gm[:, 1:2] + gm[:, 3:4]) * 0.5
    pd = jnp.concatenate(
        [(gcy - acy) / ah, (gcx - acx) / aw,
         jnp.log(gh / ah), jnp.log(gw / aw)], axis=1)
    d_ref[0] = jnp.where(cls > 0, pd, 0.0)


def _stage3(anchors, gt, ti, cls4):
    return pl.pallas_call(
        _stage3_body,
        grid=(8, NCHUNK),
        in_specs=[
            pl.BlockSpec((1, CHUNK, 4), lambda b, c: (b, c, 0)),
            pl.BlockSpec((1, G, 4), lambda b, c: (b, 0, 0)),
            pl.BlockSpec((1, 1, CHUNK, 1), lambda b, c: (b, c, 0, 0)),
            pl.BlockSpec((1, 1, CHUNK, 1), lambda b, c: (b, c, 0, 0)),
        ],
        out_specs=pl.BlockSpec((1, CHUNK, 4), lambda b, c: (b, c, 0)),
        out_shape=jax.ShapeDtypeStruct((8, A, 4), jnp.float32),
    )(anchors, gt, ti, cls4)


def kernel(anchors, true_bboxes):
    gt_t = true_bboxes.transpose(0, 2, 1)
    tc0, ti, amax = _stage1(anchors, gt_t)
    streams = jnp.asarray(_STREAMS)
    classes = _stage2(tc0.reshape(8, A), amax.reshape(8, 128), streams)
    deltas = _stage3(anchors, true_bboxes, ti.reshape(8, NCHUNK, CHUNK, 1),
                     classes.reshape(8, NCHUNK, CHUNK, 1))
    return classes, deltas


# speedup vs baseline: 6.0697x; 3.8136x over previous
import functools

import numpy as np
import jax
import jax.numpy as jnp
from jax import lax
from jax.experimental import pallas as pl
from jax.experimental.pallas import tpu as pltpu
from jax.experimental.pallas import tpu_sc as plsc

NUM_TRAIN = 256
A = 20000
AP = 20480
G = 100
CHUNKL = 2048
NCHUNK = AP // CHUNKL
NSTREAM = 16384
NEGSZ = AP + 272


def _mt_streams(nseeds: int, n: int) -> np.ndarray:
    u32 = np.uint32

    def twist(key):
        def f(cur, nxt, far):
            y = (cur & u32(0x80000000)) | (nxt & u32(0x7FFFFFFF))
            v = far ^ (y >> u32(1))
            return np.where((y & u32(1)) == 1, v ^ u32(0x9908B0DF), v)

        new = np.empty_like(key)
        new[:227] = f(key[:227], key[1:228], key[397:624])
        new[227:454] = f(key[227:454], key[228:455], new[0:227])
        new[454:623] = f(key[454:623], key[455:624], new[227:396])
        new[623] = f(key[623:624], new[0:1], new[396:397])[0]
        return new

    def temper(y):
        y = y ^ (y >> u32(11))
        y = y ^ ((y << u32(7)) & u32(0x9D2C5680))
        y = y ^ ((y << u32(15)) & u32(0xEFC60000))
        y = y ^ (y >> u32(18))
        return y

    out = np.empty((nseeds, n), dtype=np.uint32)
    nblocks = -(-n // 624)
    for seed in range(nseeds):
        key = np.empty(624, dtype=np.uint64)
        s = seed & 0xFFFFFFFF
        for p in range(624):
            key[p] = s
            s = (1812433253 * (s ^ (s >> 30)) + p + 1) & 0xFFFFFFFF
        key = key.astype(np.uint32)
        blocks = []
        for _ in range(nblocks):
            key = twist(key)
            blocks.append(temper(key))
        out[seed] = np.concatenate(blocks)[:n]
    return out.view(np.int32)


_STREAMS = _mt_streams(8, NSTREAM)


def _stagea_body(a_ref, gc_ref, gt_ref, tc_ref, am_ref, pd_ref, mx_scr, am_scr):
    c = pl.program_id(1)
    a = jnp.clip(a_ref[0], 0.0, 1.0)
    ay1, ax1, ay2, ax2 = a[0:1, :], a[1:2, :], a[2:3, :], a[3:4, :]
    g = gc_ref[0]
    gy1, gx1, gy2, gx2 = g[:, 0:1], g[:, 1:2], g[:, 2:3], g[:, 3:4]
    ih = jnp.maximum(jnp.minimum(ay2, gy2) - jnp.maximum(ay1, gy1), 0.0)
    iw = jnp.maximum(jnp.minimum(ax2, gx2) - jnp.maximum(ax1, gx1), 0.0)
    inter = ih * iw
    area_a = (ay2 - ay1) * (ax2 - ax1)
    area_g = (gy2 - gy1) * (gx2 - gx1)
    ov = inter / (area_a + area_g - inter + 1e-8)

    mx = jnp.max(ov, axis=0, keepdims=True)
    iota_g = lax.broadcasted_iota(jnp.int32, (G, CHUNKL), 0)
    ti = jnp.min(jnp.where(ov == mx, iota_g, G), axis=0, keepdims=True)
    tc = jnp.where(mx < 0.3, 0, -1).astype(jnp.int32)
    tc = jnp.where(mx >= 0.7, 1, tc)
    gidx = c * CHUNKL + lax.broadcasted_iota(jnp.int32, (1, CHUNKL), 1)
    tc_ref[0] = jnp.where(gidx < A, tc, -1)

    cmx = jnp.max(ov, axis=1, keepdims=True)
    iota_a = lax.broadcasted_iota(jnp.int32, (G, CHUNKL), 1) + c * CHUNKL
    cam = jnp.min(jnp.where(ov == cmx, iota_a, AP), axis=1, keepdims=True)

    @pl.when(c == 0)
    def _():
        mx_scr[0:G, 0:1] = jnp.full((G, 1), -1.0, jnp.float32)
        am_scr[0:G, 0:1] = jnp.zeros((G, 1), jnp.int32)

    prev_mx = mx_scr[0:G, 0:1]
    upd = cmx > prev_mx
    mx_scr[0:G, 0:1] = jnp.where(upd, cmx, prev_mx)
    am_scr[0:G, 0:1] = jnp.where(upd, cam, am_scr[0:G, 0:1])

    @pl.when(c == NCHUNK - 1)
    def _():
        am_ref[0, 0:G, :] = am_scr[0:G, 0:1]

    onehot = (iota_g == ti).astype(jnp.float32)
    gm = lax.dot_general(gt_ref[0], onehot, (((1,), (0,)), ((), ())),
                         precision=lax.Precision.HIGHEST,
                         preferred_element_type=jnp.float32)
    ah = jnp.maximum(ay2 - ay1, 1e-6)
    aw = jnp.maximum(ax2 - ax1, 1e-6)
    acy = (ay1 + ay2) * 0.5
    acx = (ax1 + ax2) * 0.5
    gh = jnp.maximum(gm[2:3, :] - gm[0:1, :], 1e-6)
    gw = jnp.maximum(gm[3:4, :] - gm[1:2, :], 1e-6)
    gcy = (gm[0:1, :] + gm[2:3, :]) * 0.5
    gcx = (gm[1:2, :] + gm[3:4, :]) * 0.5
    pd_ref[0] = jnp.concatenate(
        [(gcy - acy) / ah, (gcx - acx) / aw,
         jnp.log(gh / ah), jnp.log(gw / aw)], axis=0)


def _stagea(a_t, gt, gt_t):
    return pl.pallas_call(
        _stagea_body,
        grid=(8, NCHUNK),
        in_specs=[
            pl.BlockSpec((1, 4, CHUNKL), lambda b, c: (b, 0, c)),
            pl.BlockSpec((1, G, 4), lambda b, c: (b, 0, 0)),
            pl.BlockSpec((1, 4, G), lambda b, c: (b, 0, 0)),
        ],
        out_specs=[
            pl.BlockSpec((1, 1, CHUNKL), lambda b, c: (b, 0, c)),
            pl.BlockSpec((1, 128, 1), lambda b, c: (b, 0, 0)),
            pl.BlockSpec((1, 4, CHUNKL), lambda b, c: (b, 0, c)),
        ],
        out_shape=[
            jax.ShapeDtypeStruct((8, 1, AP), jnp.int32),
            jax.ShapeDtypeStruct((8, 128, 1), jnp.int32),
            jax.ShapeDtypeStruct((8, 4, AP), jnp.float32),
        ],
        scratch_shapes=[
            pltpu.VMEM((128, 8), jnp.float32),
            pltpu.VMEM((128, 8), jnp.int32),
        ],
    )(a_t, gt, gt_t)


def _sc_body(tc_hbm, am_hbm, st_hbm, cls_hbm, tc_v, am_v, st_v, pos_v, neg_v):
    cid = lax.axis_index("c")
    sid = lax.axis_index("s")
    wid = sid * 2 + cid

    @pl.when(wid < 8)
    def _():
        b = wid
        with jax.named_scope("sc_dma_in"):
            pltpu.sync_copy(tc_hbm.at[b], tc_v)
            pltpu.sync_copy(am_hbm.at[b], am_v)
            pltpu.sync_copy(st_hbm.at[b], st_v.at[pl.ds(0, NSTREAM)])

        iota16 = lax.iota(jnp.int32, 16)
        one16 = jnp.ones((16,), jnp.int32)

        for k in range(7):
            idx = am_v[pl.ds(k * 16, 16)]
            msk = (iota16 + k * 16) < G
            plsc.store_scatter(tc_v, [idx], one16, mask=msk)

        def _scalar(x):
            return x[0] if getattr(x, "ndim", 0) else x

        def comp_body(ci, carry):
            poff, noff = carry
            for u in range(5):
                cc = ci * 5 + u
                v = tc_v[pl.ds(cc * 16, 16)]
                ids = iota16 + cc * 16
                pm = v == 1
                nm = v == 0
                plsc.store_compressed(pos_v.at[pl.ds(poff, 16)], ids, mask=pm)
                plsc.store_compressed(neg_v.at[pl.ds(noff, 16)], ids, mask=nm)
                poff = poff + _scalar(plsc.all_reduce_population_count(pm))
                noff = noff + _scalar(plsc.all_reduce_population_count(nm))
            return (poff, noff)

        with jax.named_scope("sc_compact"):
            p, m = lax.fori_loop(0, AP // 80, comp_body,
                                 (jnp.int32(0), jnp.int32(0)))

        zero16 = jnp.zeros((16,), jnp.int32)

        def scan_accept(i, t):
            msk = i | (i >> 1)
            msk = msk | (msk >> 2)
            msk = msk | (msk >> 4)
            msk = msk | (msk >> 8)
            msk = msk | (msk >> 16)

            def cond(tt):
                v = st_v[pl.ds(tt, 16)] & msk
                return jnp.logical_not(jnp.any(v <= i)) & (tt < NSTREAM)

            t = lax.while_loop(cond, lambda tt: tt + 16, t)
            v = st_v[pl.ds(t, 16)] & msk
            r = plsc.all_reduce_ffs(v <= i)
            r = r[0] if getattr(r, "ndim", 0) else r
            j = v.at[zero16 + r].get(mode="promise_in_bounds")[0]
            return j, t + r + 1

        swap_perm = jnp.where(iota16 == 0, 1, 0)

        def shuffle_phase(buf, n, excess, t, count_tail):
            def sbody(s, t):
                i = n - 1 - s
                j, t = scan_accept(i, t)
                idxv = jnp.where(iota16 == 0, i, j)
                vals = plsc.load_gather(buf, [idxv])
                swapped = vals.at[swap_perm].get(mode="promise_in_bounds")
                plsc.store_scatter(buf, [idxv], swapped, mask=iota16 < 2)
                return t

            t = lax.fori_loop(0, jnp.where(excess > 0, n - excess, 0),
                              sbody, t)

            if not count_tail:
                return t

            def cbody(s, t):
                i = excess - 1 - s
                _, t = scan_accept(i, t)
                return t

            return lax.fori_loop(0, jnp.where(excess > 0, excess - 1, 0),
                                 cbody, t)

        excess_pos = p - NUM_TRAIN // 2
        with jax.named_scope("sc_shuffle_pos"):
            t = shuffle_phase(pos_v, p, excess_pos, jnp.int32(0), True)
        remaining_pos = p - jnp.maximum(excess_pos, 0)
        desired_neg = NUM_TRAIN - remaining_pos
        excess_neg = m - desired_neg
        with jax.named_scope("sc_shuffle_neg"):
            shuffle_phase(neg_v, m, excess_neg, t, False)

        kp = excess_pos > 0
        kn = excess_neg > 0

        neg_one16 = jnp.full((16,), -1, jnp.int32)

        def ms_body(ci, carry):
            for u in range(10):
                tc_v[pl.ds((ci * 10 + u) * 16, 16)] = neg_one16
            return carry

        with jax.named_scope("sc_memset"):
            lax.fori_loop(0, AP // 160, ms_body, jnp.int32(0))

        pstart = jnp.where(kp, excess_pos, 0)
        pcount = jnp.where(kp, NUM_TRAIN // 2, p)

        def pk_body(k, carry):
            idx = pos_v[pl.ds(pstart + k * 16, 16)]
            msk = (iota16 + k * 16) < pcount
            plsc.store_scatter(tc_v, [idx], one16, mask=msk)
            return carry

        lax.fori_loop(0, (pcount + 15) // 16, pk_body, jnp.int32(0))

        nstart = jnp.where(kn, excess_neg, 0)
        ncount = jnp.where(kn, desired_neg, m)

        def nk_body(k, carry):
            idx = neg_v[pl.ds(nstart + k * 16, 16)]
            msk = (iota16 + k * 16) < ncount
            plsc.store_scatter(tc_v, [idx], jnp.zeros((16,), jnp.int32),
                               mask=msk)
            return carry

        lax.fori_loop(0, (ncount + 15) // 16, nk_body, jnp.int32(0))

        with jax.named_scope("sc_dma_out"):
            pltpu.sync_copy(tc_v, cls_hbm.at[b])


def _stage2(tc0, amax, streams):
    mesh = plsc.VectorSubcoreMesh(core_axis_name="c", subcore_axis_name="s")
    f = functools.partial(
        pl.kernel,
        mesh=mesh,
        compiler_params=pltpu.CompilerParams(needs_layout_passes=False),
        out_type=jax.ShapeDtypeStruct((8, AP), jnp.int32),
        scratch_types=[
            pltpu.VMEM((AP,), jnp.int32),
            pltpu.VMEM((128,), jnp.int32),
            pltpu.VMEM((NSTREAM + 16,), jnp.int32),
            pltpu.VMEM((AP + 16,), jnp.int32),
            pltpu.VMEM((NEGSZ,), jnp.int32),
        ],
    )(_sc_body)
    return f(tc0, amax, streams)


def _stageb_body(pd_ref, cls_ref, d_ref):
    d_ref[0] = jnp.where(cls_ref[0] == 1, pd_ref[0], 0.0)


def _stageb(pdt, cls3):
    return pl.pallas_call(
        _stageb_body,
        grid=(8, NCHUNK),
        in_specs=[
            pl.BlockSpec((1, 4, CHUNKL), lambda b, c: (b, 0, c)),
            pl.BlockSpec((1, 1, CHUNKL), lambda b, c: (b, 0, c)),
        ],
        out_specs=pl.BlockSpec((1, 4, CHUNKL), lambda b, c: (b, 0, c)),
        out_shape=jax.ShapeDtypeStruct((8, 4, AP), jnp.float32),
    )(pdt, cls3)


def kernel(anchors, true_bboxes):
    a_pad = jnp.concatenate(
        [anchors, jnp.zeros((8, AP - A, 4), jnp.float32)], axis=1)
    a_t = a_pad.transpose(0, 2, 1)
    gt_t = true_bboxes.transpose(0, 2, 1)
    tc0, amax, pdt = _stagea(a_t, true_bboxes, gt_t)
    streams = jnp.asarray(_STREAMS)
    classes_full = _stage2(tc0.reshape(8, AP), amax.reshape(8, 128), streams)
    deltas_t = _stageb(pdt, classes_full.reshape(8, 1, AP))
    classes = classes_full[:, :A]
    deltas = deltas_t[:, :, :A].transpose(0, 2, 1)
    return classes, deltas
